# baseline (speedup 1.0000x reference)
import functools

import jax
import jax.numpy as jnp
from jax import lax
from jax.experimental import pallas as pl
from jax.experimental.pallas import tpu as pltpu
from jax.experimental.pallas import tpu_sc as plsc

_NUM_CLASSES = 1000000
_EMBED_DIM = 64
_BATCH = 16384

_info = plsc.get_sparse_core_info()
_NC, _NS = _info.num_cores, _info.num_subcores
_NW = _NC * _NS
_B_PER_W = _BATCH // _NW
_WAVE = 16
_NWAVE = _B_PER_W // _WAVE
_DEPTH = 3 * _WAVE

_mesh = plsc.VectorSubcoreMesh(core_axis_name="c", subcore_axis_name="s")


@functools.partial(
    pl.kernel,
    mesh=_mesh,
    out_type=jax.ShapeDtypeStruct((_BATCH, _EMBED_DIM), jnp.float32),
    scratch_types=[
        pltpu.VMEM((_B_PER_W + 16,), jnp.int32),
        pltpu.VMEM((_DEPTH, 8, _EMBED_DIM), jnp.float32),
        pltpu.VMEM((_B_PER_W, _EMBED_DIM), jnp.float32),
        pltpu.SemaphoreType.DMA,
        pltpu.SemaphoreType.DMA,
        pltpu.SemaphoreType.DMA,
    ],
    compiler_params=pltpu.CompilerParams(
        use_tc_tiling_on_sc=True, needs_layout_passes=False
    ),
)
def _sc_gather(idx_hbm, tbl_hbm, out_hbm, idx_v, ring_v, rows_v, s0, s1, s2):
    wid = lax.axis_index("s") * _NC + lax.axis_index("c")
    base = wid * _B_PER_W
    pltpu.sync_copy(
        idx_hbm.at[pl.ds(base, _B_PER_W)], idx_v.at[pl.ds(0, _B_PER_W)]
    )
    lane = lax.iota(jnp.int32, 16)
    sems = (s0, s1, s2)

    def fetch_wave(m, buf, sem):
        vec = idx_v[pl.ds(m * _WAVE, 16)]
        for l in range(_WAVE):
            pltpu.async_copy(
                tbl_hbm.at[vec[l] // 8], ring_v.at[buf * _WAVE + l], sem
            )

    def drain_wave(buf, sem):
        pltpu.make_async_copy(
            tbl_hbm.at[pl.ds(0, _WAVE)],
            ring_v.at[pl.ds(buf * _WAVE, _WAVE)],
            sem,
        ).wait()

    def extract_wave(m, buf):
        vec = idx_v[pl.ds(m * _WAVE, 16)]
        for l in range(_WAVE):
            c = m * _WAVE + l
            r = vec[l] % 8
            face = ring_v.at[buf * _WAVE + l, r]
            for k in range(_EMBED_DIM // 16):
                rows_v[c, pl.ds(k * 16, 16)] = face[pl.ds(k * 16, 16)]

    for b in range(3):
        fetch_wave(b, b, sems[b])

    def triple(t, _):
        m0 = 3 * t
        for b in range(3):
            drain_wave(b, sems[b])
            extract_wave(m0 + b, b)

            @pl.when(m0 + b + 3 < _NWAVE)
            def _(m=m0 + b + 3, b=b):
                fetch_wave(m, b, sems[b])

        return 0

    lax.fori_loop(0, _NWAVE // 3, triple, 0)
    for b in range(_NWAVE % 3):
        drain_wave(b, sems[b])
        extract_wave(_NWAVE - (_NWAVE % 3) + b, b)
    pltpu.sync_copy(rows_v, out_hbm.at[pl.ds(base, _B_PER_W)])


def kernel(class_labels, embedding):
    tbl3 = embedding[:_NUM_CLASSES].reshape(_NUM_CLASSES // 8, 8, _EMBED_DIM)
    out = _sc_gather(class_labels.astype(jnp.int32), tbl3)
    return out[:, None, :]
